# baseline (speedup 1.0000x reference)
import jax
import jax.numpy as jnp
from jax import lax
from jax.experimental import pallas as pl
from jax.experimental.pallas import tpu as pltpu

N = 1024
IN = 256
OUT = 128
NH = 4
HD = OUT // NH
BLK = 256
G = N // BLK


def _sem_kernel(x_ref, adj_ref, wlin_ref, wlay_ref, watt_ref, out_ref,
                u_scr, v_scr, xw_scr, cnt_scr):
    i = pl.program_id(0)

    @pl.when(i == 0)
    def _prologue():
        x = x_ref[...]
        h = lax.dot_general(x, wlin_ref[...], (((1,), (1,)), ((), ())),
                            preferred_element_type=jnp.float32)
        watt = watt_ref[...].reshape(NH, 2 * OUT)
        a = watt[:, :OUT]
        b = watt[:, OUT:]
        s = lax.dot_general(h, a, (((1,), (1,)), ((), ())),
                            preferred_element_type=jnp.float32)
        t = lax.dot_general(b, h, (((1,), (1,)), ((), ())),
                            preferred_element_type=jnp.float32)
        u_scr[...] = jnp.exp(-jnp.clip(s, -30.0, 30.0)).astype(jnp.bfloat16)
        v_scr[...] = jnp.exp(-jnp.clip(t, -30.0, 30.0)).astype(jnp.bfloat16)
        wlay = wlay_ref[...].reshape(OUT, IN)
        xw_scr[...] = lax.dot_general(x, wlay, (((1,), (1,)), ((), ())),
                                      preferred_element_type=jnp.float32)
        cnt_scr[0] = 0.0

    adjb = adj_ref[...]
    zero = (adjb == 0.0)
    cnt_scr[0] += jnp.sum(jnp.where(zero, 1.0, 0.0))

    xw = xw_scr[...]
    row = pl.ds(i * BLK, BLK)
    for k in range(NH):
        uk = u_scr[row, k:k + 1]
        vk = v_scr[k:k + 1, :]
        one = jnp.bfloat16(1.0)
        sig = one / (one + uk * vk)
        akb = jnp.where(zero, jnp.bfloat16(0.0), sig)
        xwk = xw[:, k * HD:(k + 1) * HD].astype(jnp.bfloat16)
        ok = lax.dot_general(akb, xwk, (((1,), (0,)), ((), ())),
                             preferred_element_type=jnp.float32)
        out_ref[row, k * HD:(k + 1) * HD] = ok

    @pl.when(i == G - 1)
    def _pad_fix():
        pad = cnt_scr[0]
        for k in range(NH):
            e00 = 1.0 / (1.0 + u_scr[0:1, k:k + 1].astype(jnp.float32)
                         * v_scr[k:k + 1, 0:1].astype(jnp.float32))
            cs = slice(k * HD, (k + 1) * HD)
            out_ref[0:1, cs] = out_ref[0:1, cs] + (pad * e00) * xw_scr[0:1, cs]


@jax.jit
def kernel(x, adj, W_lin, W_layers, W_atts, W_c1, W_c2):
    del W_c1, W_c2
    return pl.pallas_call(
        _sem_kernel,
        grid=(G,),
        in_specs=[
            pl.BlockSpec((N, IN), lambda i: (0, 0)),
            pl.BlockSpec((BLK, N), lambda i: (i, 0)),
            pl.BlockSpec((OUT, IN), lambda i: (0, 0)),
            pl.BlockSpec((NH, HD, IN), lambda i: (0, 0, 0)),
            pl.BlockSpec((NH, 1, 2 * OUT), lambda i: (0, 0, 0)),
        ],
        out_specs=pl.BlockSpec((N, OUT), lambda i: (0, 0)),
        scratch_shapes=[
            pltpu.VMEM((N, NH), jnp.bfloat16),
            pltpu.VMEM((NH, N), jnp.bfloat16),
            pltpu.VMEM((N, OUT), jnp.float32),
            pltpu.SMEM((1,), jnp.float32),
        ],
        out_shape=jax.ShapeDtypeStruct((N, OUT), jnp.float32),
    )(x, adj, W_lin, W_layers, W_atts)


# speedup vs baseline: 1.0170x; 1.0170x over previous
import jax
import jax.numpy as jnp
from jax import lax
from jax.experimental import pallas as pl
from jax.experimental.pallas import tpu as pltpu

N = 1024
IN = 256
OUT = 128
NH = 4
HD = OUT // NH
BLK = 256
G = N // BLK


def _sem_kernel(x_ref, adj_ref, wlin_ref, wlay_ref, watt_ref, out_ref,
                u_scr, v_scr, xw_scr, cnt_scr):
    i = pl.program_id(0)

    @pl.when(i == 0)
    def _prologue():
        x = x_ref[...]
        h = lax.dot_general(x, wlin_ref[...], (((1,), (1,)), ((), ())),
                            preferred_element_type=jnp.float32)
        watt = watt_ref[...].reshape(NH, 2 * OUT)
        a = watt[:, :OUT]
        b = watt[:, OUT:]
        s = lax.dot_general(h, a, (((1,), (1,)), ((), ())),
                            preferred_element_type=jnp.float32)
        t = lax.dot_general(b, h, (((1,), (1,)), ((), ())),
                            preferred_element_type=jnp.float32)
        u_scr[...] = jnp.exp(-jnp.clip(s, -30.0, 30.0))
        v_scr[...] = jnp.exp(-jnp.clip(t, -30.0, 30.0))
        wlay = wlay_ref[...].reshape(OUT, IN)
        xw_scr[...] = lax.dot_general(x, wlay, (((1,), (1,)), ((), ())),
                                      preferred_element_type=jnp.float32)
        cnt_scr[0] = 0.0

    adjb = adj_ref[...]
    zero = (adjb == 0.0)
    cnt_scr[0] += jnp.sum(jnp.where(zero, 1.0, 0.0))

    xw = xw_scr[...]
    row = pl.ds(i * BLK, BLK)
    for k in range(NH):
        uk = u_scr[row, k:k + 1]
        vk = v_scr[k:k + 1, :]
        sig = 1.0 / (1.0 + uk * vk)
        akb = jnp.where(zero, 0.0, sig).astype(jnp.bfloat16)
        xwk = xw[:, k * HD:(k + 1) * HD].astype(jnp.bfloat16)
        ok = lax.dot_general(akb, xwk, (((1,), (0,)), ((), ())),
                             preferred_element_type=jnp.float32)
        out_ref[row, k * HD:(k + 1) * HD] = ok

    @pl.when(i == G - 1)
    def _pad_fix():
        pad = cnt_scr[0]
        for k in range(NH):
            e00 = 1.0 / (1.0 + u_scr[0:1, k:k + 1] * v_scr[k:k + 1, 0:1])
            cs = slice(k * HD, (k + 1) * HD)
            out_ref[0:1, cs] = out_ref[0:1, cs] + (pad * e00) * xw_scr[0:1, cs]


@jax.jit
def kernel(x, adj, W_lin, W_layers, W_atts, W_c1, W_c2):
    del W_c1, W_c2
    return pl.pallas_call(
        _sem_kernel,
        grid=(G,),
        in_specs=[
            pl.BlockSpec((N, IN), lambda i: (0, 0)),
            pl.BlockSpec((BLK, N), lambda i: (i, 0)),
            pl.BlockSpec((OUT, IN), lambda i: (0, 0)),
            pl.BlockSpec((NH, HD, IN), lambda i: (0, 0, 0)),
            pl.BlockSpec((NH, 1, 2 * OUT), lambda i: (0, 0, 0)),
        ],
        out_specs=pl.BlockSpec((N, OUT), lambda i: (0, 0)),
        scratch_shapes=[
            pltpu.VMEM((N, NH), jnp.float32),
            pltpu.VMEM((NH, N), jnp.float32),
            pltpu.VMEM((N, OUT), jnp.float32),
            pltpu.SMEM((1,), jnp.float32),
        ],
        out_shape=jax.ShapeDtypeStruct((N, OUT), jnp.float32),
    )(x, adj, W_lin, W_layers, W_atts)


# speedup vs baseline: 1.1229x; 1.1041x over previous
import jax
import jax.numpy as jnp
from jax import lax
from jax.experimental import pallas as pl
from jax.experimental.pallas import tpu as pltpu

N = 1024
IN = 256
OUT = 128
NH = 4
HD = OUT // NH
BLK = 512
G = N // BLK


def _sem_kernel(x_ref, adj_ref, wlin_ref, wlay_ref, watt_ref, out_ref,
                u_scr, v_scr, xw_scr, cnt_scr):
    i = pl.program_id(0)

    @pl.when(i == 0)
    def _prologue():
        x = x_ref[...]
        h = lax.dot_general(x, wlin_ref[...], (((1,), (1,)), ((), ())),
                            preferred_element_type=jnp.float32)
        watt = watt_ref[...].reshape(NH, 2 * OUT)
        a = watt[:, :OUT]
        b = watt[:, OUT:]
        s = lax.dot_general(h, a, (((1,), (1,)), ((), ())),
                            preferred_element_type=jnp.float32)
        t = lax.dot_general(b, h, (((1,), (1,)), ((), ())),
                            preferred_element_type=jnp.float32)
        u_scr[...] = jnp.exp(-jnp.clip(s, -30.0, 30.0))
        v_scr[...] = jnp.exp(-jnp.clip(t, -30.0, 30.0))
        wlay = wlay_ref[...].reshape(OUT, IN)
        xw_scr[...] = lax.dot_general(x, wlay, (((1,), (1,)), ((), ())),
                                      preferred_element_type=jnp.float32)
        cnt_scr[0] = 0.0

    adjb = adj_ref[...]
    zero = (adjb == 0.0)
    cnt_scr[0] += jnp.sum(jnp.where(zero, 1.0, 0.0))

    xw = xw_scr[...]
    row = pl.ds(i * BLK, BLK)
    for k in range(NH):
        uk = u_scr[row, k:k + 1]
        vk = v_scr[k:k + 1, :]
        sig = 1.0 / (1.0 + uk * vk)
        akb = jnp.where(zero, 0.0, sig).astype(jnp.bfloat16)
        xwk = xw[:, k * HD:(k + 1) * HD].astype(jnp.bfloat16)
        ok = lax.dot_general(akb, xwk, (((1,), (0,)), ((), ())),
                             preferred_element_type=jnp.float32)
        out_ref[row, k * HD:(k + 1) * HD] = ok

    @pl.when(i == G - 1)
    def _pad_fix():
        pad = cnt_scr[0]
        for k in range(NH):
            e00 = 1.0 / (1.0 + u_scr[0:1, k:k + 1] * v_scr[k:k + 1, 0:1])
            cs = slice(k * HD, (k + 1) * HD)
            out_ref[0:1, cs] = out_ref[0:1, cs] + (pad * e00) * xw_scr[0:1, cs]


@jax.jit
def kernel(x, adj, W_lin, W_layers, W_atts, W_c1, W_c2):
    del W_c1, W_c2
    return pl.pallas_call(
        _sem_kernel,
        grid=(G,),
        in_specs=[
            pl.BlockSpec((N, IN), lambda i: (0, 0)),
            pl.BlockSpec((BLK, N), lambda i: (i, 0)),
            pl.BlockSpec((OUT, IN), lambda i: (0, 0)),
            pl.BlockSpec((NH, HD, IN), lambda i: (0, 0, 0)),
            pl.BlockSpec((NH, 1, 2 * OUT), lambda i: (0, 0, 0)),
        ],
        out_specs=pl.BlockSpec((N, OUT), lambda i: (0, 0)),
        scratch_shapes=[
            pltpu.VMEM((N, NH), jnp.float32),
            pltpu.VMEM((NH, N), jnp.float32),
            pltpu.VMEM((N, OUT), jnp.float32),
            pltpu.SMEM((1,), jnp.float32),
        ],
        out_shape=jax.ShapeDtypeStruct((N, OUT), jnp.float32),
    )(x, adj, W_lin, W_layers, W_atts)


# speedup vs baseline: 1.1473x; 1.0217x over previous
import jax
import jax.numpy as jnp
from jax import lax
from jax.experimental import pallas as pl
from jax.experimental.pallas import tpu as pltpu

N = 1024
IN = 256
OUT = 128
NH = 4
HD = OUT // NH
BLK = 1024
G = N // BLK


def _sem_kernel(x_ref, adj_ref, wlin_ref, wlay_ref, watt_ref, out_ref,
                u_scr, v_scr, xw_scr, cnt_scr):
    i = pl.program_id(0)

    @pl.when(i == 0)
    def _prologue():
        x = x_ref[...]
        h = lax.dot_general(x, wlin_ref[...], (((1,), (1,)), ((), ())),
                            preferred_element_type=jnp.float32)
        watt = watt_ref[...].reshape(NH, 2 * OUT)
        a = watt[:, :OUT]
        b = watt[:, OUT:]
        s = lax.dot_general(h, a, (((1,), (1,)), ((), ())),
                            preferred_element_type=jnp.float32)
        t = lax.dot_general(b, h, (((1,), (1,)), ((), ())),
                            preferred_element_type=jnp.float32)
        u_scr[...] = jnp.exp(-jnp.clip(s, -30.0, 30.0))
        v_scr[...] = jnp.exp(-jnp.clip(t, -30.0, 30.0))
        wlay = wlay_ref[...].reshape(OUT, IN)
        xw_scr[...] = lax.dot_general(x, wlay, (((1,), (1,)), ((), ())),
                                      preferred_element_type=jnp.float32)
        cnt_scr[0] = 0.0

    adjb = adj_ref[...]
    zero = (adjb == 0.0)
    cnt_scr[0] += jnp.sum(jnp.where(zero, 1.0, 0.0))

    xw = xw_scr[...]
    row = pl.ds(i * BLK, BLK)
    for k in range(NH):
        uk = u_scr[row, k:k + 1]
        vk = v_scr[k:k + 1, :]
        sig = 1.0 / (1.0 + uk * vk)
        akb = jnp.where(zero, 0.0, sig).astype(jnp.bfloat16)
        xwk = xw[:, k * HD:(k + 1) * HD].astype(jnp.bfloat16)
        ok = lax.dot_general(akb, xwk, (((1,), (0,)), ((), ())),
                             preferred_element_type=jnp.float32)
        out_ref[row, k * HD:(k + 1) * HD] = ok

    @pl.when(i == G - 1)
    def _pad_fix():
        pad = cnt_scr[0]
        for k in range(NH):
            e00 = 1.0 / (1.0 + u_scr[0:1, k:k + 1] * v_scr[k:k + 1, 0:1])
            cs = slice(k * HD, (k + 1) * HD)
            out_ref[0:1, cs] = out_ref[0:1, cs] + (pad * e00) * xw_scr[0:1, cs]


@jax.jit
def kernel(x, adj, W_lin, W_layers, W_atts, W_c1, W_c2):
    del W_c1, W_c2
    return pl.pallas_call(
        _sem_kernel,
        grid=(G,),
        in_specs=[
            pl.BlockSpec((N, IN), lambda i: (0, 0)),
            pl.BlockSpec((BLK, N), lambda i: (i, 0)),
            pl.BlockSpec((OUT, IN), lambda i: (0, 0)),
            pl.BlockSpec((NH, HD, IN), lambda i: (0, 0, 0)),
            pl.BlockSpec((NH, 1, 2 * OUT), lambda i: (0, 0, 0)),
        ],
        out_specs=pl.BlockSpec((N, OUT), lambda i: (0, 0)),
        scratch_shapes=[
            pltpu.VMEM((N, NH), jnp.float32),
            pltpu.VMEM((NH, N), jnp.float32),
            pltpu.VMEM((N, OUT), jnp.float32),
            pltpu.SMEM((1,), jnp.float32),
        ],
        out_shape=jax.ShapeDtypeStruct((N, OUT), jnp.float32),
    )(x, adj, W_lin, W_layers, W_atts)
